# baseline (speedup 1.0000x reference)
import functools

import jax
import jax.numpy as jnp
from jax import lax
from jax.experimental import pallas as pl
from jax.experimental.pallas import tpu as pltpu
from jax.experimental.pallas import tpu_sc as plsc

N = 10000
E = 160000
F_IN = 256
H = 512

NC = 2
NS = 16
LC = 128
NCHUNK = H // LC

BK = 128
EPAD = 163840
NPAD = 10112
ROWS_PER_TILE = NPAD // NS
BM = 1000

_MESH = plsc.VectorSubcoreMesh(
    core_axis_name="c", subcore_axis_name="s", num_cores=NC, num_subcores=NS)


def _deg_body(e_blk, ones_hbm, zrows, deg_out, acc, ones_v, eidx):
  cid = lax.axis_index("c")
  sid = lax.axis_index("s")
  row0 = sid * ROWS_PER_TILE
  nblk = EPAD // (NC * NS * BK)

  pltpu.sync_copy(ones_hbm, ones_v)
  pltpu.sync_copy(zrows, acc.at[pl.ds(row0, ROWS_PER_TILE)])
  pltpu.sync_copy(e_blk.at[sid, pl.ds(cid * nblk, nblk)], eidx)
  plsc.subcore_barrier()

  def body(b, carry):
    pltpu.sync_copy(ones_v, acc.at[eidx.at[b, 1]], add=True)
    return carry

  lax.fori_loop(0, nblk, body, 0)
  plsc.subcore_barrier()

  @pl.when(cid == 0)
  def _():
    pltpu.sync_copy(acc.at[pl.ds(row0, ROWS_PER_TILE)],
                    deg_out.at[0, pl.ds(row0, ROWS_PER_TILE)])

  @pl.when(cid == 1)
  def _():
    pltpu.sync_copy(acc.at[pl.ds(row0, ROWS_PER_TILE)],
                    deg_out.at[1, pl.ds(row0, ROWS_PER_TILE)])


_deg_call = functools.partial(
    pl.kernel,
    out_type=jax.ShapeDtypeStruct((NC, NPAD, LC), jnp.float32),
    mesh=_MESH,
    scratch_types=[
        pltpu.VMEM_SHARED((NPAD, LC), jnp.float32),
        pltpu.VMEM((BK, LC), jnp.float32),
        pltpu.VMEM((EPAD // (NC * NS * BK), 2, BK), jnp.int32),
    ],
)(_deg_body)


NBLK = EPAD // (NS * BK)
HB = NBLK // 2


def _prop_body(g0, g1, g2, g3, e_blk, zrows, acc_out,
               acc, eidx, rows0, rows1, sem0, sem1):
  cid = lax.axis_index("c")
  sid = lax.axis_index("s")
  row0 = sid * ROWS_PER_TILE
  rows = (rows0, rows1)
  sems = (sem0, sem1)

  def do_half(tbl, half):
    pltpu.sync_copy(e_blk.at[sid, pl.ds(half * HB, HB)], eidx)
    for j in range(2):
      pltpu.async_copy(tbl.at[eidx.at[j, 0]], rows[j], sems[j])

    def grp(bg, carry):
      for j in range(2):
        b = 2 * bg + j
        pltpu.make_async_copy(tbl.at[eidx.at[b, 0]], rows[j], sems[j]).wait()
        pltpu.sync_copy(rows[j], acc.at[eidx.at[b, 1]], add=True)
        pltpu.async_copy(tbl.at[eidx.at[b + 2, 0]], rows[j], sems[j])
      return carry

    lax.fori_loop(0, HB // 2 - 1, grp, 0)
    for j in range(2):
      b = HB - 2 + j
      pltpu.make_async_copy(tbl.at[eidx.at[b, 0]], rows[j], sems[j]).wait()
      pltpu.sync_copy(rows[j], acc.at[eidx.at[b, 1]], add=True)

  def do_chunk(tbl, c):
    pltpu.sync_copy(zrows, acc.at[pl.ds(row0, ROWS_PER_TILE)])
    plsc.subcore_barrier()
    for half in range(2):
      do_half(tbl, half)
    plsc.subcore_barrier()
    pltpu.sync_copy(acc.at[pl.ds(row0, ROWS_PER_TILE)],
                    acc_out.at[c, pl.ds(row0, ROWS_PER_TILE)])
    plsc.subcore_barrier()

  @pl.when(cid == 0)
  def _():
    do_chunk(g0, 0)
    do_chunk(g2, 2)

  @pl.when(cid == 1)
  def _():
    do_chunk(g1, 1)
    do_chunk(g3, 3)


_prop_call = functools.partial(
    pl.kernel,
    out_type=jax.ShapeDtypeStruct((NCHUNK, NPAD, LC), jnp.float32),
    mesh=_MESH,
    compiler_params=pltpu.CompilerParams(use_tc_tiling_on_sc=False),
    scratch_types=[
        pltpu.VMEM_SHARED((NPAD, LC), jnp.float32),
        pltpu.VMEM((HB, 2, BK), jnp.int32),
        pltpu.VMEM((BK, LC), jnp.float32),
        pltpu.VMEM((BK, LC), jnp.float32),
        pltpu.SemaphoreType.DMA,
        pltpu.SemaphoreType.DMA,
    ],
)(_prop_body)


def _dv(deg_ref):
  return lax.rsqrt(deg_ref[0] + deg_ref[1] + 1.0)


def _mm_body(x_ref, w_ref, deg_ref, g0, g1, g2, g3):
  h = jnp.dot(x_ref[...], w_ref[...], preferred_element_type=jnp.float32)
  dv = _dv(deg_ref)
  for c, gc in enumerate((g0, g1, g2, g3)):
    gc[...] = dv * h[:, c * LC:(c + 1) * LC]


def _mm_call(xin, w, degp):
  f = xin.shape[1]
  gspec = pl.BlockSpec((BM, LC), lambda i: (i, 0))
  gshape = jax.ShapeDtypeStruct((N, LC), jnp.float32)
  return pl.pallas_call(
      _mm_body,
      grid=(N // BM,),
      in_specs=[
          pl.BlockSpec((BM, f), lambda i: (i, 0)),
          pl.BlockSpec((f, H), lambda i: (0, 0)),
          pl.BlockSpec((NC, BM, LC), lambda i: (0, i, 0)),
      ],
      out_specs=[gspec, gspec, gspec, gspec],
      out_shape=[gshape, gshape, gshape, gshape],
  )(xin, w, degp)


def _fmm_body(acc_ref, g0, g1, g2, g3, deg_ref, b_ref, gm_ref, bt_ref,
              w_ref, o0, o1, o2, o3):
  dv = _dv(deg_ref)
  zs = []
  for c, gc in enumerate((g0, g1, g2, g3)):
    o = dv * (acc_ref[c] + gc[...]) + b_ref[c]
    zs.append(jnp.maximum(o, 0.0) * gm_ref[c] + bt_ref[c])
  z = jnp.concatenate(zs, axis=1)
  h = jnp.dot(z, w_ref[...], preferred_element_type=jnp.float32)
  for c, oc in enumerate((o0, o1, o2, o3)):
    oc[...] = dv * h[:, c * LC:(c + 1) * LC]


def _fmm_call(acc, gs, degp, b, gm, bt, w):
  cspec = pl.BlockSpec((BM, LC), lambda i: (i, 0))
  pspec = pl.BlockSpec((NCHUNK, LC), lambda i: (0, 0))
  gshape = jax.ShapeDtypeStruct((N, LC), jnp.float32)
  return pl.pallas_call(
      _fmm_body,
      grid=(N // BM,),
      in_specs=[
          pl.BlockSpec((NCHUNK, BM, LC), lambda i: (0, i, 0)),
          cspec, cspec, cspec, cspec,
          pl.BlockSpec((NC, BM, LC), lambda i: (0, i, 0)),
          pspec, pspec, pspec,
          pl.BlockSpec((H, H), lambda i: (0, 0)),
      ],
      out_specs=[cspec, cspec, cspec, cspec],
      out_shape=[gshape, gshape, gshape, gshape],
  )(acc, *gs, degp, b, gm, bt, w)


def _comb_body(acc_ref, g0, g1, g2, g3, deg_ref, b_ref, z_ref):
  dv = _dv(deg_ref)
  for c, gc in enumerate((g0, g1, g2, g3)):
    z_ref[:, c * LC:(c + 1) * LC] = dv * (acc_ref[c] + gc[...]) + b_ref[c]


def _comb_call(acc, gs, degp, b):
  cspec = pl.BlockSpec((BM, LC), lambda i: (i, 0))
  return pl.pallas_call(
      _comb_body,
      grid=(N // BM,),
      in_specs=[
          pl.BlockSpec((NCHUNK, BM, LC), lambda i: (0, i, 0)),
          cspec, cspec, cspec, cspec,
          pl.BlockSpec((NC, BM, LC), lambda i: (0, i, 0)),
          pl.BlockSpec((NCHUNK, LC), lambda i: (0, 0)),
      ],
      out_specs=pl.BlockSpec((BM, H), lambda i: (i, 0)),
      out_shape=jax.ShapeDtypeStruct((N, H), jnp.float32),
  )(acc, *gs, degp, b)


def kernel(x, edge_index, batch, params, W1, b1, W2, b2, W3, b3, W4, b4,
           g1, be1, g2, be2, g3, be3, W_emb, b_emb):
  pad = EPAD - E
  srcp = jnp.concatenate([edge_index[0], jnp.zeros((pad,), jnp.int32)])
  dstp = jnp.concatenate([edge_index[1], jnp.full((pad,), N, jnp.int32)])
  e_blk = jnp.stack([srcp.reshape(NS, NBLK, BK), dstp.reshape(NS, NBLK, BK)],
                    axis=2)
  zrows = jnp.zeros((ROWS_PER_TILE, LC), jnp.float32)
  ones128 = jnp.ones((BK, LC), jnp.float32)

  degp = _deg_call(e_blk, ones128, zrows)

  gs = _mm_call(x, W1, degp)
  acc = _prop_call(*gs, e_blk, zrows)
  for b, gm, bt, w in ((b1, g1, be1, W2), (b2, g2, be2, W3), (b3, g3, be3, W4)):
    gs = _fmm_call(acc, gs, degp, b.reshape(NCHUNK, LC),
                   gm.reshape(NCHUNK, LC), bt.reshape(NCHUNK, LC), w)
    acc = _prop_call(*gs, e_blk, zrows)
  return _comb_call(acc, gs, degp, b4.reshape(NCHUNK, LC))


# speedup vs baseline: 1.0770x; 1.0770x over previous
import functools

import jax
import jax.numpy as jnp
from jax import lax
from jax.experimental import pallas as pl
from jax.experimental.pallas import tpu as pltpu
from jax.experimental.pallas import tpu_sc as plsc

N = 10000
E = 160000
F_IN = 256
H = 512

NC = 2
NS = 16
LC = 128
NCHUNK = H // LC

BK = 128
EPAD = 163840
NPAD = 10112
ROWS_PER_TILE = NPAD // NS
BM = 1000

_MESH = plsc.VectorSubcoreMesh(
    core_axis_name="c", subcore_axis_name="s", num_cores=NC, num_subcores=NS)


def _deg_body(e_blk, ones_hbm, zrows, deg_out, acc, ones_v, eidx):
  cid = lax.axis_index("c")
  sid = lax.axis_index("s")
  row0 = sid * ROWS_PER_TILE
  nblk = EPAD // (NC * NS * BK)

  pltpu.sync_copy(ones_hbm, ones_v)
  pltpu.sync_copy(zrows, acc.at[pl.ds(row0, ROWS_PER_TILE)])
  pltpu.sync_copy(e_blk.at[sid, pl.ds(cid * nblk, nblk)], eidx)
  plsc.subcore_barrier()

  def body(b, carry):
    pltpu.sync_copy(ones_v, acc.at[eidx.at[b, 1]], add=True)
    return carry

  lax.fori_loop(0, nblk, body, 0)
  plsc.subcore_barrier()

  @pl.when(cid == 0)
  def _():
    pltpu.sync_copy(acc.at[pl.ds(row0, ROWS_PER_TILE)],
                    deg_out.at[0, pl.ds(row0, ROWS_PER_TILE)])

  @pl.when(cid == 1)
  def _():
    pltpu.sync_copy(acc.at[pl.ds(row0, ROWS_PER_TILE)],
                    deg_out.at[1, pl.ds(row0, ROWS_PER_TILE)])


_deg_call = functools.partial(
    pl.kernel,
    out_type=jax.ShapeDtypeStruct((NC, NPAD, LC), jnp.float32),
    mesh=_MESH,
    scratch_types=[
        pltpu.VMEM_SHARED((NPAD, LC), jnp.float32),
        pltpu.VMEM((BK, LC), jnp.float32),
        pltpu.VMEM((EPAD // (NC * NS * BK), 2, BK), jnp.int32),
    ],
)(_deg_body)


NBLK = EPAD // (NS * BK)
HB = NBLK // 2


def _prop_body(g0, g1, g2, g3, e_blk, zrows, acc_out,
               acc, eidx, rows0, rows1, sem0, sem1):
  cid = lax.axis_index("c")
  sid = lax.axis_index("s")
  row0 = sid * ROWS_PER_TILE
  rows = (rows0, rows1)
  sems = (sem0, sem1)

  def do_half(tbl, half):
    pltpu.sync_copy(e_blk.at[sid, pl.ds(half * HB, HB)], eidx)
    for j in range(2):
      pltpu.async_copy(tbl.at[eidx.at[j, 0]], rows[j], sems[j])

    def grp(bg, carry):
      for j in range(2):
        b = 2 * bg + j
        pltpu.make_async_copy(tbl.at[eidx.at[b, 0]], rows[j], sems[j]).wait()
        pltpu.sync_copy(rows[j], acc.at[eidx.at[b, 1]], add=True)
        pltpu.async_copy(tbl.at[eidx.at[b + 2, 0]], rows[j], sems[j])
      return carry

    lax.fori_loop(0, HB // 2 - 1, grp, 0)
    for j in range(2):
      b = HB - 2 + j
      pltpu.make_async_copy(tbl.at[eidx.at[b, 0]], rows[j], sems[j]).wait()
      pltpu.sync_copy(rows[j], acc.at[eidx.at[b, 1]], add=True)

  def do_chunk(tbl, c):
    pltpu.sync_copy(zrows, acc.at[pl.ds(row0, ROWS_PER_TILE)])
    plsc.subcore_barrier()
    for half in range(2):
      do_half(tbl, half)
    plsc.subcore_barrier()
    pltpu.sync_copy(acc.at[pl.ds(row0, ROWS_PER_TILE)],
                    acc_out.at[c, pl.ds(row0, ROWS_PER_TILE)])
    plsc.subcore_barrier()

  @pl.when(cid == 0)
  def _():
    do_chunk(g0, 0)
    do_chunk(g2, 2)

  @pl.when(cid == 1)
  def _():
    do_chunk(g1, 1)
    do_chunk(g3, 3)


_prop_call = functools.partial(
    pl.kernel,
    out_type=jax.ShapeDtypeStruct((NCHUNK, NPAD, LC), jnp.float32),
    mesh=_MESH,
    scratch_types=[
        pltpu.VMEM_SHARED((NPAD, LC), jnp.float32),
        pltpu.VMEM((HB, 2, BK), jnp.int32),
        pltpu.VMEM((BK, LC), jnp.float32),
        pltpu.VMEM((BK, LC), jnp.float32),
        pltpu.SemaphoreType.DMA,
        pltpu.SemaphoreType.DMA,
    ],
)(_prop_body)


def _dv(deg_ref):
  return lax.rsqrt(deg_ref[0] + deg_ref[1] + 1.0)


def _mm_body(x_ref, w_ref, deg_ref, g0, g1, g2, g3):
  h = jnp.dot(x_ref[...], w_ref[...], preferred_element_type=jnp.float32)
  dv = _dv(deg_ref)
  for c, gc in enumerate((g0, g1, g2, g3)):
    gc[...] = dv * h[:, c * LC:(c + 1) * LC]


def _mm_call(xin, w, degp):
  f = xin.shape[1]
  gspec = pl.BlockSpec((BM, LC), lambda i: (i, 0))
  gshape = jax.ShapeDtypeStruct((N, LC), jnp.float32)
  return pl.pallas_call(
      _mm_body,
      grid=(N // BM,),
      in_specs=[
          pl.BlockSpec((BM, f), lambda i: (i, 0)),
          pl.BlockSpec((f, H), lambda i: (0, 0)),
          pl.BlockSpec((NC, BM, LC), lambda i: (0, i, 0)),
      ],
      out_specs=[gspec, gspec, gspec, gspec],
      out_shape=[gshape, gshape, gshape, gshape],
  )(xin, w, degp)


def _fmm_body(acc_ref, g0, g1, g2, g3, deg_ref, b_ref, gm_ref, bt_ref,
              w_ref, o0, o1, o2, o3):
  dv = _dv(deg_ref)
  zs = []
  for c, gc in enumerate((g0, g1, g2, g3)):
    o = dv * (acc_ref[c] + gc[...]) + b_ref[c]
    zs.append(jnp.maximum(o, 0.0) * gm_ref[c] + bt_ref[c])
  z = jnp.concatenate(zs, axis=1)
  h = jnp.dot(z, w_ref[...], preferred_element_type=jnp.float32)
  for c, oc in enumerate((o0, o1, o2, o3)):
    oc[...] = dv * h[:, c * LC:(c + 1) * LC]


def _fmm_call(acc, gs, degp, b, gm, bt, w):
  cspec = pl.BlockSpec((BM, LC), lambda i: (i, 0))
  pspec = pl.BlockSpec((NCHUNK, LC), lambda i: (0, 0))
  gshape = jax.ShapeDtypeStruct((N, LC), jnp.float32)
  return pl.pallas_call(
      _fmm_body,
      grid=(N // BM,),
      in_specs=[
          pl.BlockSpec((NCHUNK, BM, LC), lambda i: (0, i, 0)),
          cspec, cspec, cspec, cspec,
          pl.BlockSpec((NC, BM, LC), lambda i: (0, i, 0)),
          pspec, pspec, pspec,
          pl.BlockSpec((H, H), lambda i: (0, 0)),
      ],
      out_specs=[cspec, cspec, cspec, cspec],
      out_shape=[gshape, gshape, gshape, gshape],
  )(acc, *gs, degp, b, gm, bt, w)


def _comb_body(acc_ref, g0, g1, g2, g3, deg_ref, b_ref, z_ref):
  dv = _dv(deg_ref)
  for c, gc in enumerate((g0, g1, g2, g3)):
    z_ref[:, c * LC:(c + 1) * LC] = dv * (acc_ref[c] + gc[...]) + b_ref[c]


def _comb_call(acc, gs, degp, b):
  cspec = pl.BlockSpec((BM, LC), lambda i: (i, 0))
  return pl.pallas_call(
      _comb_body,
      grid=(N // BM,),
      in_specs=[
          pl.BlockSpec((NCHUNK, BM, LC), lambda i: (0, i, 0)),
          cspec, cspec, cspec, cspec,
          pl.BlockSpec((NC, BM, LC), lambda i: (0, i, 0)),
          pl.BlockSpec((NCHUNK, LC), lambda i: (0, 0)),
      ],
      out_specs=pl.BlockSpec((BM, H), lambda i: (i, 0)),
      out_shape=jax.ShapeDtypeStruct((N, H), jnp.float32),
  )(acc, *gs, degp, b)


def kernel(x, edge_index, batch, params, W1, b1, W2, b2, W3, b3, W4, b4,
           g1, be1, g2, be2, g3, be3, W_emb, b_emb):
  pad = EPAD - E
  srcp = jnp.concatenate([edge_index[0], jnp.zeros((pad,), jnp.int32)])
  dstp = jnp.concatenate([edge_index[1], jnp.full((pad,), N, jnp.int32)])
  e_blk = jnp.stack([srcp.reshape(NS, NBLK, BK), dstp.reshape(NS, NBLK, BK)],
                    axis=2)
  zrows = jnp.zeros((ROWS_PER_TILE, LC), jnp.float32)
  ones128 = jnp.ones((BK, LC), jnp.float32)

  degp = _deg_call(e_blk, ones128, zrows)

  gs = _mm_call(x, W1, degp)
  acc = _prop_call(*gs, e_blk, zrows)
  for b, gm, bt, w in ((b1, g1, be1, W2), (b2, g2, be2, W3), (b3, g3, be3, W4)):
    gs = _fmm_call(acc, gs, degp, b.reshape(NCHUNK, LC),
                   gm.reshape(NCHUNK, LC), bt.reshape(NCHUNK, LC), w)
    acc = _prop_call(*gs, e_blk, zrows)
  return _comb_call(acc, gs, degp, b4.reshape(NCHUNK, LC))
